# Initial kernel scaffold; baseline (speedup 1.0000x reference)
import jax
import jax.numpy as jnp
from jax.experimental import pallas as pl


def kernel(scores_map, descriptor_map):
    raise NotImplementedError("write your pallas kernel here")


# speedup vs baseline: 1.0159x; 1.0159x over previous
import functools

import jax
import jax.numpy as jnp
from jax import lax
from jax.experimental import pallas as pl
from jax.experimental.pallas import tpu as pltpu

_RADIUS = 2
_TOP_K = 4096
_NMS_R = 3
_H = 512
_W = 512
_HALF = _H // 2


def _shift_max(a, r, axis):
    n = a.shape[axis]
    idx = lax.broadcasted_iota(jnp.int32, a.shape, axis)
    out = a
    for d in range(1, r + 1):
        up = jnp.roll(a, -d, axis=axis)
        up = jnp.where(idx < n - d, up, -jnp.inf)
        dn = jnp.roll(a, d, axis=axis)
        dn = jnp.where(idx >= d, dn, -jnp.inf)
        out = jnp.maximum(out, jnp.maximum(up, dn))
    return out


def _pool(a, r):
    return _shift_max(_shift_max(a, r, 0), r, 1)


def _nms_body(x_ref, o_ref):
    half = pl.program_id(1)
    s = x_ref[0, 0]
    mp = _pool(s, _NMS_R)
    max_mask = s == mp
    for _ in range(2):
        supp_mask = _pool(max_mask.astype(jnp.float32), _NMS_R) > 0
        supp_scores = jnp.where(supp_mask, 0.0, s)
        new_max = supp_scores == _pool(supp_scores, _NMS_R)
        max_mask = max_mask | (new_max & (~supp_mask))
    out = jnp.where(max_mask, s, 0.0)
    row = lax.broadcasted_iota(jnp.int32, s.shape, 0) + half * _HALF
    col = lax.broadcasted_iota(jnp.int32, s.shape, 1)
    keep = ((row > _RADIUS) & (row < _H - _RADIUS)
            & (col > _RADIUS) & (col < _W - _RADIUS))
    o_ref[0] = jnp.where(keep, out, 0.0)


def _nms_pallas(scores_map):
    b = scores_map.shape[0]
    return pl.pallas_call(
        _nms_body,
        grid=(b, 2),
        in_specs=[pl.BlockSpec((1, 1, _HALF, _W), lambda i, j: (i, 0, j, 0))],
        out_specs=pl.BlockSpec((1, _HALF, _W), lambda i, j: (i, j, 0)),
        out_shape=jax.ShapeDtypeStruct((b, _H, _W), jnp.float32),
    )(scores_map)


def _grid_sample_bilinear(img, kxy):
    H, W = img.shape
    x = (kxy[:, 0] + 1.0) * 0.5 * (W - 1)
    y = (kxy[:, 1] + 1.0) * 0.5 * (H - 1)
    x0 = jnp.floor(x)
    y0 = jnp.floor(y)
    wx1 = x - x0
    wx0 = 1.0 - wx1
    wy1 = y - y0
    wy0 = 1.0 - wy1
    x0i = jnp.clip(x0, 0, W - 1).astype(jnp.int32)
    x1i = jnp.clip(x0 + 1, 0, W - 1).astype(jnp.int32)
    y0i = jnp.clip(y0, 0, H - 1).astype(jnp.int32)
    y1i = jnp.clip(y0 + 1, 0, H - 1).astype(jnp.int32)
    return (wy0 * wx0 * img[y0i, x0i] + wy0 * wx1 * img[y0i, x1i]
            + wy1 * wx0 * img[y1i, x0i] + wy1 * wx1 * img[y1i, x1i])


def kernel(scores_map, descriptor_map):
    b, _, h, w = scores_map.shape
    nms = _nms_pallas(scores_map)
    flat = nms.reshape(b, -1)
    _, idx = lax.top_k(flat, _TOP_K)
    kx = (idx % w).astype(jnp.float32)
    ky = (idx // w).astype(jnp.float32)
    kxy = jnp.stack([kx, ky], axis=-1)
    denom = jnp.array([w - 1, h - 1], dtype=jnp.float32)
    kxy = kxy / denom * 2.0 - 1.0
    kptscores = jax.vmap(_grid_sample_bilinear)(scores_map[:, 0], kxy)

    B, C, H, W = descriptor_map.shape
    scale = jnp.array([W - 1, H - 1], dtype=jnp.float32)

    def one(dm, k):
        ki = ((k + 1.0) / 2.0 * scale).astype(jnp.int32)
        d = dm[:, ki[:, 1], ki[:, 0]]
        n = jnp.sqrt(jnp.sum(d * d, axis=0, keepdims=True))
        d = d / jnp.maximum(n, 1e-12)
        return d.T

    descriptors = jax.vmap(one)(descriptor_map, kxy)
    return kxy, descriptors, kptscores


# speedup vs baseline: 3.6386x; 3.5819x over previous
import functools

import jax
import jax.numpy as jnp
from jax import lax
from jax.experimental import pallas as pl
from jax.experimental.pallas import tpu as pltpu
from jax.experimental.pallas import tpu_sc as plsc

_RADIUS = 2
_TOP_K = 4096
_NMS_R = 3
_H = 512
_W = 512
_HALF = _H // 2

_N_SUB = 16
_SEG = (_H * _W) // _N_SUB
_LANE_SEG = _SEG // 16
_CAPL = 128
_CAP = 16 * _CAPL
_FILL = (_RADIUS + 1) * _W


def _shift_max(a, r, axis):
    n = a.shape[axis]
    idx = lax.broadcasted_iota(jnp.int32, a.shape, axis)
    out = a
    for d in range(1, r + 1):
        up = jnp.roll(a, -d, axis=axis)
        up = jnp.where(idx < n - d, up, -jnp.inf)
        dn = jnp.roll(a, d, axis=axis)
        dn = jnp.where(idx >= d, dn, -jnp.inf)
        out = jnp.maximum(out, jnp.maximum(up, dn))
    return out


def _pool(a, r):
    return _shift_max(_shift_max(a, r, 0), r, 1)


def _nms_body(x_ref, o_ref):
    half = pl.program_id(1)
    s = x_ref[0, 0]
    mp = _pool(s, _NMS_R)
    max_mask = s == mp
    for _ in range(2):
        supp_mask = _pool(max_mask.astype(jnp.float32), _NMS_R) > 0
        supp_scores = jnp.where(supp_mask, 0.0, s)
        new_max = supp_scores == _pool(supp_scores, _NMS_R)
        max_mask = max_mask | (new_max & (~supp_mask))
    out = jnp.where(max_mask, s, 0.0)
    row = lax.broadcasted_iota(jnp.int32, s.shape, 0) + half * _HALF
    col = lax.broadcasted_iota(jnp.int32, s.shape, 1)
    keep = ((row > _RADIUS) & (row < _H - _RADIUS)
            & (col > _RADIUS) & (col < _W - _RADIUS))
    o_ref[0] = jnp.where(keep, out, 0.0)


def _nms_pallas(scores_map):
    b = scores_map.shape[0]
    return pl.pallas_call(
        _nms_body,
        grid=(b, 2),
        in_specs=[pl.BlockSpec((1, 1, _HALF, _W), lambda i, j: (i, 0, j, 0))],
        out_specs=pl.BlockSpec((1, _HALF, _W), lambda i, j: (i, j, 0)),
        out_shape=jax.ShapeDtypeStruct((b, _H, _W), jnp.float32),
    )(scores_map)


def _compact_body(nms_hbm, vals_hbm, idx_hbm, buf_v, vals_v, idxs_v):
    c = lax.axis_index("c")
    s = lax.axis_index("s")
    pltpu.sync_copy(nms_hbm.at[c, pl.ds(s * _SEG, _SEG)], buf_v)

    def fill(j, _):
        vals_v[pl.ds(j * 16, 16)] = jnp.full((16,), -1.0, jnp.float32)
        idxs_v[pl.ds(j * 16, 16)] = jnp.zeros((16,), jnp.int32)
        return 0

    lax.fori_loop(0, (_CAP + 16) // 16, fill, 0)
    lane = lax.iota(jnp.int32, 16)
    one = jnp.full((16,), 1, jnp.int32)
    zero = jnp.zeros((16,), jnp.int32)
    lane_base = lane * _CAPL
    gbase = lane * _LANE_SEG
    base0 = s * _SEG

    def step(i, cnt):
        gidx = gbase + i
        v = plsc.load_gather(buf_v, [gidx])
        fidx = base0 + gidx
        m = v != 0.0
        dst = jnp.where(m & (cnt < _CAPL), lane_base + cnt, _CAP)
        plsc.store_scatter(vals_v, [dst], v)
        plsc.store_scatter(idxs_v, [dst], fidx)
        return cnt + jnp.where(m, one, zero)

    lax.fori_loop(0, _LANE_SEG, step, zero)
    pltpu.sync_copy(vals_v.at[pl.ds(0, _CAP)], vals_hbm.at[c, s])
    pltpu.sync_copy(idxs_v.at[pl.ds(0, _CAP)], idx_hbm.at[c, s])


def _compact_sc(nms_flat):
    b = nms_flat.shape[0]
    run = functools.partial(
        pl.kernel,
        mesh=plsc.VectorSubcoreMesh(core_axis_name="c", subcore_axis_name="s"),
        compiler_params=pltpu.CompilerParams(needs_layout_passes=False),
        out_type=(
            jax.ShapeDtypeStruct((b, _N_SUB, _CAP), jnp.float32),
            jax.ShapeDtypeStruct((b, _N_SUB, _CAP), jnp.int32),
        ),
        scratch_types=[
            pltpu.VMEM((_SEG,), jnp.float32),
            pltpu.VMEM((_CAP + 16,), jnp.float32),
            pltpu.VMEM((_CAP + 16,), jnp.int32),
        ],
    )(_compact_body)
    return run(nms_flat)


def _grid_sample_bilinear(img, kxy):
    H, W = img.shape
    x = (kxy[:, 0] + 1.0) * 0.5 * (W - 1)
    y = (kxy[:, 1] + 1.0) * 0.5 * (H - 1)
    x0 = jnp.floor(x)
    y0 = jnp.floor(y)
    wx1 = x - x0
    wx0 = 1.0 - wx1
    wy1 = y - y0
    wy0 = 1.0 - wy1
    x0i = jnp.clip(x0, 0, W - 1).astype(jnp.int32)
    x1i = jnp.clip(x0 + 1, 0, W - 1).astype(jnp.int32)
    y0i = jnp.clip(y0, 0, H - 1).astype(jnp.int32)
    y1i = jnp.clip(y0 + 1, 0, H - 1).astype(jnp.int32)
    return (wy0 * wx0 * img[y0i, x0i] + wy0 * wx1 * img[y0i, x1i]
            + wy1 * wx0 * img[y1i, x0i] + wy1 * wx1 * img[y1i, x1i])


def kernel(scores_map, descriptor_map):
    b, _, h, w = scores_map.shape
    nms = _nms_pallas(scores_map)
    cvals, cidx = _compact_sc(nms.reshape(b, -1))
    fill_v = jnp.zeros((b, _FILL), jnp.float32)
    fill_i = jnp.broadcast_to(jnp.arange(_FILL, dtype=jnp.int32), (b, _FILL))
    allv = jnp.concatenate([fill_v, cvals.reshape(b, -1)], axis=1)
    alli = jnp.concatenate([fill_i, cidx.reshape(b, -1)], axis=1)
    _, pos = lax.top_k(allv, _TOP_K)
    idx = jnp.take_along_axis(alli, pos, axis=1)
    kx = (idx % w).astype(jnp.float32)
    ky = (idx // w).astype(jnp.float32)
    kxy = jnp.stack([kx, ky], axis=-1)
    denom = jnp.array([w - 1, h - 1], dtype=jnp.float32)
    kxy = kxy / denom * 2.0 - 1.0
    kptscores = jax.vmap(_grid_sample_bilinear)(scores_map[:, 0], kxy)

    B, C, H, W = descriptor_map.shape
    scale = jnp.array([W - 1, H - 1], dtype=jnp.float32)

    def one(dm, k):
        ki = ((k + 1.0) / 2.0 * scale).astype(jnp.int32)
        d = dm[:, ki[:, 1], ki[:, 0]]
        n = jnp.sqrt(jnp.sum(d * d, axis=0, keepdims=True))
        d = d / jnp.maximum(n, 1e-12)
        return d.T

    descriptors = jax.vmap(one)(descriptor_map, kxy)
    return kxy, descriptors, kptscores


# speedup vs baseline: 3.6747x; 1.0099x over previous
import functools

import jax
import jax.numpy as jnp
from jax import lax
from jax.experimental import pallas as pl
from jax.experimental.pallas import tpu as pltpu
from jax.experimental.pallas import tpu_sc as plsc

_RADIUS = 2
_TOP_K = 4096
_NMS_R = 3
_H = 512
_W = 512
_HALF = _H // 2

_N_SUB = 16
_SEG = (_H * _W) // _N_SUB
_LANE_SEG = _SEG // 16
_CAPL = 128
_CAP = 16 * _CAPL
_FILL = (_RADIUS + 1) * _W


def _shift_max(a, r, axis):
    n = a.shape[axis]
    idx = lax.broadcasted_iota(jnp.int32, a.shape, axis)
    out = a
    for d in range(1, r + 1):
        up = jnp.roll(a, -d, axis=axis)
        up = jnp.where(idx < n - d, up, -jnp.inf)
        dn = jnp.roll(a, d, axis=axis)
        dn = jnp.where(idx >= d, dn, -jnp.inf)
        out = jnp.maximum(out, jnp.maximum(up, dn))
    return out


def _pool(a, r):
    return _shift_max(_shift_max(a, r, 0), r, 1)


def _nms_body(x_ref, o_ref):
    half = pl.program_id(1)
    s = x_ref[0, 0]
    mp = _pool(s, _NMS_R)
    max_mask = s == mp
    for _ in range(2):
        supp_mask = _pool(max_mask.astype(jnp.float32), _NMS_R) > 0
        supp_scores = jnp.where(supp_mask, 0.0, s)
        new_max = supp_scores == _pool(supp_scores, _NMS_R)
        max_mask = max_mask | (new_max & (~supp_mask))
    out = jnp.where(max_mask, s, 0.0)
    row = lax.broadcasted_iota(jnp.int32, s.shape, 0) + half * _HALF
    col = lax.broadcasted_iota(jnp.int32, s.shape, 1)
    keep = ((row > _RADIUS) & (row < _H - _RADIUS)
            & (col > _RADIUS) & (col < _W - _RADIUS))
    o_ref[0] = jnp.where(keep, out, 0.0)


def _nms_pallas(scores_map):
    b = scores_map.shape[0]
    return pl.pallas_call(
        _nms_body,
        grid=(b, 2),
        in_specs=[pl.BlockSpec((1, 1, _HALF, _W), lambda i, j: (i, 0, j, 0))],
        out_specs=pl.BlockSpec((1, _HALF, _W), lambda i, j: (i, j, 0)),
        out_shape=jax.ShapeDtypeStruct((b, _H, _W), jnp.float32),
    )(scores_map)


def _compact_body(nms_hbm, vals_hbm, idx_hbm, buf_v, vals_v, idxs_v):
    c = lax.axis_index("c")
    s = lax.axis_index("s")
    pltpu.sync_copy(nms_hbm.at[c, pl.ds(s * _SEG, _SEG)], buf_v)

    def fill(j, _):
        vals_v[pl.ds(j * 16, 16)] = jnp.full((16,), -1.0, jnp.float32)
        idxs_v[pl.ds(j * 16, 16)] = jnp.zeros((16,), jnp.int32)
        return 0

    lax.fori_loop(0, (_CAP + 16) // 16, fill, 0)
    lane = lax.iota(jnp.int32, 16)
    one = jnp.full((16,), 1, jnp.int32)
    zero = jnp.zeros((16,), jnp.int32)
    lane_base = lane * _CAPL
    gbase = lane * _LANE_SEG
    base0 = s * _SEG

    def step(i, cnt):
        gidx = gbase + i
        v = plsc.load_gather(buf_v, [gidx])
        fidx = base0 + gidx
        m = v != 0.0
        dst = jnp.where(m & (cnt < _CAPL), lane_base + cnt, _CAP)
        plsc.store_scatter(vals_v, [dst], v)
        plsc.store_scatter(idxs_v, [dst], fidx)
        return cnt + jnp.where(m, one, zero)

    lax.fori_loop(0, _LANE_SEG, step, zero)
    pltpu.sync_copy(vals_v.at[pl.ds(0, _CAP)], vals_hbm.at[c, s])
    pltpu.sync_copy(idxs_v.at[pl.ds(0, _CAP)], idx_hbm.at[c, s])


def _compact_sc(nms_flat):
    b = nms_flat.shape[0]
    run = functools.partial(
        pl.kernel,
        mesh=plsc.VectorSubcoreMesh(core_axis_name="c", subcore_axis_name="s"),
        compiler_params=pltpu.CompilerParams(needs_layout_passes=False),
        out_type=(
            jax.ShapeDtypeStruct((b, _N_SUB, _CAP), jnp.float32),
            jax.ShapeDtypeStruct((b, _N_SUB, _CAP), jnp.int32),
        ),
        scratch_types=[
            pltpu.VMEM((_SEG,), jnp.float32),
            pltpu.VMEM((_CAP + 16,), jnp.float32),
            pltpu.VMEM((_CAP + 16,), jnp.int32),
        ],
    )(_compact_body)
    return run(nms_flat)


_N_KPT = _TOP_K
_CH = 96
_CH_PER_SUB = _CH // _N_SUB


def _desc_gather_body(dm_hbm, idx_hbm, out_hbm, idx_v, absidx_v, row_v, sem):
    c = lax.axis_index("c")
    s = lax.axis_index("s")
    pltpu.sync_copy(idx_hbm.at[c], idx_v)
    for ch in range(_CH_PER_SUB):
        chan = s * _CH_PER_SUB + ch
        base = (c * _CH + chan) * (_H * _W)

        def add(j, _):
            absidx_v[pl.ds(j * 16, 16)] = idx_v[pl.ds(j * 16, 16)] + base
            return 0

        lax.fori_loop(0, _N_KPT // 16, add, 0)
        pltpu.async_copy(dm_hbm.at[absidx_v], row_v, sem).wait()
        pltpu.sync_copy(row_v, out_hbm.at[c, chan])


def _desc_gather_sc(dm_flat, gidx):
    b = gidx.shape[0]
    run = functools.partial(
        pl.kernel,
        mesh=plsc.VectorSubcoreMesh(core_axis_name="c", subcore_axis_name="s"),
        compiler_params=pltpu.CompilerParams(needs_layout_passes=False),
        out_type=jax.ShapeDtypeStruct((b, _CH, _N_KPT), jnp.float32),
        scratch_types=[
            pltpu.VMEM((_N_KPT,), jnp.int32),
            pltpu.VMEM((_N_KPT,), jnp.int32),
            pltpu.VMEM((_N_KPT,), jnp.float32),
            pltpu.SemaphoreType.DMA,
        ],
    )(_desc_gather_body)
    return run(dm_flat, gidx)


def _grid_sample_bilinear(img, kxy):
    H, W = img.shape
    x = (kxy[:, 0] + 1.0) * 0.5 * (W - 1)
    y = (kxy[:, 1] + 1.0) * 0.5 * (H - 1)
    x0 = jnp.floor(x)
    y0 = jnp.floor(y)
    wx1 = x - x0
    wx0 = 1.0 - wx1
    wy1 = y - y0
    wy0 = 1.0 - wy1
    x0i = jnp.clip(x0, 0, W - 1).astype(jnp.int32)
    x1i = jnp.clip(x0 + 1, 0, W - 1).astype(jnp.int32)
    y0i = jnp.clip(y0, 0, H - 1).astype(jnp.int32)
    y1i = jnp.clip(y0 + 1, 0, H - 1).astype(jnp.int32)
    return (wy0 * wx0 * img[y0i, x0i] + wy0 * wx1 * img[y0i, x1i]
            + wy1 * wx0 * img[y1i, x0i] + wy1 * wx1 * img[y1i, x1i])


def kernel(scores_map, descriptor_map):
    b, _, h, w = scores_map.shape
    nms = _nms_pallas(scores_map)
    cvals, cidx = _compact_sc(nms.reshape(b, -1))
    fill_v = jnp.zeros((b, _FILL), jnp.float32)
    fill_i = jnp.broadcast_to(jnp.arange(_FILL, dtype=jnp.int32), (b, _FILL))
    allv = jnp.concatenate([fill_v, cvals.reshape(b, -1)], axis=1)
    alli = jnp.concatenate([fill_i, cidx.reshape(b, -1)], axis=1)
    _, pos = lax.top_k(allv, _TOP_K)
    idx = jnp.take_along_axis(alli, pos, axis=1)
    kx = (idx % w).astype(jnp.float32)
    ky = (idx // w).astype(jnp.float32)
    kxy = jnp.stack([kx, ky], axis=-1)
    denom = jnp.array([w - 1, h - 1], dtype=jnp.float32)
    kxy = kxy / denom * 2.0 - 1.0
    kptscores = jax.vmap(_grid_sample_bilinear)(scores_map[:, 0], kxy)

    B, C, H, W = descriptor_map.shape
    scale = jnp.array([W - 1, H - 1], dtype=jnp.float32)
    ki = ((kxy + 1.0) / 2.0 * scale).astype(jnp.int32)
    gidx = ki[:, :, 1] * W + ki[:, :, 0]
    d = _desc_gather_sc(descriptor_map.reshape(-1), gidx)
    n = jnp.sqrt(jnp.sum(d * d, axis=1, keepdims=True))
    d = d / jnp.maximum(n, 1e-12)
    descriptors = jnp.transpose(d, (0, 2, 1))
    return kxy, descriptors, kptscores


# speedup vs baseline: 4.1377x; 1.1260x over previous
import functools

import jax
import jax.numpy as jnp
from jax import lax
from jax.experimental import pallas as pl
from jax.experimental.pallas import tpu as pltpu
from jax.experimental.pallas import tpu_sc as plsc

_RADIUS = 2
_TOP_K = 4096
_NMS_R = 3
_H = 512
_W = 512
_HALF = _H // 2

_N_SUB = 16
_SEG = (_H * _W) // _N_SUB
_LANE_SEG = _SEG // 16
_CAPL = 128
_CAP = 16 * _CAPL
_FILL = (_RADIUS + 1) * _W


def _shift_max(a, r, axis):
    n = a.shape[axis]
    idx = lax.broadcasted_iota(jnp.int32, a.shape, axis)
    out = a
    for d in range(1, r + 1):
        up = jnp.roll(a, -d, axis=axis)
        up = jnp.where(idx < n - d, up, -jnp.inf)
        dn = jnp.roll(a, d, axis=axis)
        dn = jnp.where(idx >= d, dn, -jnp.inf)
        out = jnp.maximum(out, jnp.maximum(up, dn))
    return out


def _pool(a, r):
    return _shift_max(_shift_max(a, r, 0), r, 1)


def _nms_body(x_ref, o_ref):
    half = pl.program_id(1)
    s = x_ref[0, 0]
    mp = _pool(s, _NMS_R)
    max_mask = s == mp
    for _ in range(2):
        supp_mask = _pool(max_mask.astype(jnp.float32), _NMS_R) > 0
        supp_scores = jnp.where(supp_mask, 0.0, s)
        new_max = supp_scores == _pool(supp_scores, _NMS_R)
        max_mask = max_mask | (new_max & (~supp_mask))
    out = jnp.where(max_mask, s, 0.0)
    row = lax.broadcasted_iota(jnp.int32, s.shape, 0) + half * _HALF
    col = lax.broadcasted_iota(jnp.int32, s.shape, 1)
    keep = ((row > _RADIUS) & (row < _H - _RADIUS)
            & (col > _RADIUS) & (col < _W - _RADIUS))
    o_ref[0] = jnp.where(keep, out, 0.0)


def _nms_pallas(scores_map):
    b = scores_map.shape[0]
    return pl.pallas_call(
        _nms_body,
        grid=(b, 2),
        in_specs=[pl.BlockSpec((1, 1, _HALF, _W), lambda i, j: (i, 0, j, 0))],
        out_specs=pl.BlockSpec((1, _HALF, _W), lambda i, j: (i, j, 0)),
        out_shape=jax.ShapeDtypeStruct((b, _H, _W), jnp.float32),
    )(scores_map)


def _compact_body(nms_hbm, vals_hbm, idx_hbm, buf_v, vals_v, idxs_v):
    c = lax.axis_index("c")
    s = lax.axis_index("s")
    pltpu.sync_copy(nms_hbm.at[c, pl.ds(s * _SEG, _SEG)], buf_v)

    def fill(j, _):
        vals_v[pl.ds(j * 16, 16)] = jnp.full((16,), -1.0, jnp.float32)
        idxs_v[pl.ds(j * 16, 16)] = jnp.zeros((16,), jnp.int32)
        return 0

    lax.fori_loop(0, (_CAP + 16) // 16, fill, 0)
    lane = lax.iota(jnp.int32, 16)
    one = jnp.full((16,), 1, jnp.int32)
    zero = jnp.zeros((16,), jnp.int32)
    lane_base = lane * _CAPL
    gbase = lane * _LANE_SEG
    base0 = s * _SEG

    def step(i, cnt):
        gidx = gbase + i
        v = plsc.load_gather(buf_v, [gidx])
        fidx = base0 + gidx
        m = v != 0.0
        dst = jnp.where(m & (cnt < _CAPL), lane_base + cnt, _CAP)
        plsc.store_scatter(vals_v, [dst], v)
        plsc.store_scatter(idxs_v, [dst], fidx)
        return cnt + jnp.where(m, one, zero)

    lax.fori_loop(0, _LANE_SEG, step, zero)
    pltpu.sync_copy(vals_v.at[pl.ds(0, _CAP)], vals_hbm.at[c, s])
    pltpu.sync_copy(idxs_v.at[pl.ds(0, _CAP)], idx_hbm.at[c, s])


def _compact_sc(nms_flat):
    b = nms_flat.shape[0]
    run = functools.partial(
        pl.kernel,
        mesh=plsc.VectorSubcoreMesh(core_axis_name="c", subcore_axis_name="s"),
        compiler_params=pltpu.CompilerParams(needs_layout_passes=False),
        out_type=(
            jax.ShapeDtypeStruct((b, _N_SUB, _CAP), jnp.float32),
            jax.ShapeDtypeStruct((b, _N_SUB, _CAP), jnp.int32),
        ),
        scratch_types=[
            pltpu.VMEM((_SEG,), jnp.float32),
            pltpu.VMEM((_CAP + 16,), jnp.float32),
            pltpu.VMEM((_CAP + 16,), jnp.int32),
        ],
    )(_compact_body)
    return run(nms_flat)


_N_KPT = _TOP_K
_CH = 96
_CH_PER_SUB = _CH // _N_SUB


def _desc_gather_body(dm_hbm, idx_hbm, out_hbm, idx_v, absidx_v, row_v, sem):
    c = lax.axis_index("c")
    s = lax.axis_index("s")
    pltpu.sync_copy(idx_hbm.at[c], idx_v)
    for ch in range(_CH_PER_SUB):
        chan = s * _CH_PER_SUB + ch
        base = (c * _CH + chan) * (_H * _W)

        def add(j, _):
            absidx_v[pl.ds(j * 16, 16)] = idx_v[pl.ds(j * 16, 16)] + base
            return 0

        lax.fori_loop(0, _N_KPT // 16, add, 0)
        pltpu.async_copy(dm_hbm.at[absidx_v], row_v, sem).wait()
        pltpu.sync_copy(row_v, out_hbm.at[c, chan])


def _desc_gather_sc(dm_flat, gidx):
    b = gidx.shape[0]
    run = functools.partial(
        pl.kernel,
        mesh=plsc.VectorSubcoreMesh(core_axis_name="c", subcore_axis_name="s"),
        compiler_params=pltpu.CompilerParams(needs_layout_passes=False),
        out_type=jax.ShapeDtypeStruct((b, _CH, _N_KPT), jnp.float32),
        scratch_types=[
            pltpu.VMEM((_N_KPT,), jnp.int32),
            pltpu.VMEM((_N_KPT,), jnp.int32),
            pltpu.VMEM((_N_KPT,), jnp.float32),
            pltpu.SemaphoreType.DMA,
        ],
    )(_desc_gather_body)
    return run(dm_flat, gidx)


def _grid_sample_bilinear(img, kxy):
    H, W = img.shape
    x = (kxy[:, 0] + 1.0) * 0.5 * (W - 1)
    y = (kxy[:, 1] + 1.0) * 0.5 * (H - 1)
    x0 = jnp.floor(x)
    y0 = jnp.floor(y)
    wx1 = x - x0
    wx0 = 1.0 - wx1
    wy1 = y - y0
    wy0 = 1.0 - wy1
    x0i = jnp.clip(x0, 0, W - 1).astype(jnp.int32)
    x1i = jnp.clip(x0 + 1, 0, W - 1).astype(jnp.int32)
    y0i = jnp.clip(y0, 0, H - 1).astype(jnp.int32)
    y1i = jnp.clip(y0 + 1, 0, H - 1).astype(jnp.int32)
    return (wy0 * wx0 * img[y0i, x0i] + wy0 * wx1 * img[y0i, x1i]
            + wy1 * wx0 * img[y1i, x0i] + wy1 * wx1 * img[y1i, x1i])


def kernel(scores_map, descriptor_map):
    b, _, h, w = scores_map.shape
    nms = _nms_pallas(scores_map)
    cvals, cidx = _compact_sc(nms.reshape(b, -1))
    fill_v = jnp.zeros((b, _FILL), jnp.float32)
    fill_i = jnp.broadcast_to(jnp.arange(_FILL, dtype=jnp.int32), (b, _FILL))
    allv = jnp.concatenate([fill_v, cvals.reshape(b, -1)], axis=1)
    alli = jnp.concatenate([fill_i, cidx.reshape(b, -1)], axis=1)
    _, srt = lax.sort((-allv, alli), dimension=1, num_keys=1, is_stable=True)
    idx = srt[:, :_TOP_K]
    kx = (idx % w).astype(jnp.float32)
    ky = (idx // w).astype(jnp.float32)
    kxy = jnp.stack([kx, ky], axis=-1)
    denom = jnp.array([w - 1, h - 1], dtype=jnp.float32)
    kxy = kxy / denom * 2.0 - 1.0
    kptscores = jax.vmap(_grid_sample_bilinear)(scores_map[:, 0], kxy)

    B, C, H, W = descriptor_map.shape
    scale = jnp.array([W - 1, H - 1], dtype=jnp.float32)
    ki = ((kxy + 1.0) / 2.0 * scale).astype(jnp.int32)
    gidx = ki[:, :, 1] * W + ki[:, :, 0]
    d = _desc_gather_sc(descriptor_map.reshape(-1), gidx)
    n = jnp.sqrt(jnp.sum(d * d, axis=1, keepdims=True))
    d = d / jnp.maximum(n, 1e-12)
    descriptors = jnp.transpose(d, (0, 2, 1))
    return kxy, descriptors, kptscores


# speedup vs baseline: 5.9925x; 1.4482x over previous
import functools

import jax
import jax.numpy as jnp
from jax import lax
from jax.experimental import pallas as pl
from jax.experimental.pallas import tpu as pltpu
from jax.experimental.pallas import tpu_sc as plsc

_RADIUS = 2
_TOP_K = 4096
_NMS_R = 3
_H = 512
_W = 512
_HALF = _H // 2

_N_SUB = 16
_SEG = (_H * _W) // _N_SUB
_LANE_SEG = _SEG // 16
_CAPL = 64
_CAP = 16 * _CAPL
_FILL = (_RADIUS + 1) * _W


def _shift_max(a, r, axis):
    n = a.shape[axis]
    idx = lax.broadcasted_iota(jnp.int32, a.shape, axis)
    out = a
    for d in range(1, r + 1):
        up = jnp.roll(a, -d, axis=axis)
        up = jnp.where(idx < n - d, up, -jnp.inf)
        dn = jnp.roll(a, d, axis=axis)
        dn = jnp.where(idx >= d, dn, -jnp.inf)
        out = jnp.maximum(out, jnp.maximum(up, dn))
    return out


def _pool(a, r):
    return _shift_max(_shift_max(a, r, 0), r, 1)


def _nms_body(x_ref, o_ref):
    half = pl.program_id(1)
    s = x_ref[0, 0]
    mp = _pool(s, _NMS_R)
    max_mask = s == mp
    for _ in range(2):
        supp_mask = _pool(max_mask.astype(jnp.float32), _NMS_R) > 0
        supp_scores = jnp.where(supp_mask, 0.0, s)
        new_max = supp_scores == _pool(supp_scores, _NMS_R)
        max_mask = max_mask | (new_max & (~supp_mask))
    out = jnp.where(max_mask, s, 0.0)
    row = lax.broadcasted_iota(jnp.int32, s.shape, 0) + half * _HALF
    col = lax.broadcasted_iota(jnp.int32, s.shape, 1)
    keep = ((row > _RADIUS) & (row < _H - _RADIUS)
            & (col > _RADIUS) & (col < _W - _RADIUS))
    o_ref[0] = jnp.where(keep, out, 0.0)


def _nms_pallas(scores_map):
    b = scores_map.shape[0]
    return pl.pallas_call(
        _nms_body,
        grid=(b, 2),
        in_specs=[pl.BlockSpec((1, 1, _HALF, _W), lambda i, j: (i, 0, j, 0))],
        out_specs=pl.BlockSpec((1, _HALF, _W), lambda i, j: (i, j, 0)),
        out_shape=jax.ShapeDtypeStruct((b, _H, _W), jnp.float32),
    )(scores_map)


def _compact_body(nms_hbm, vals_hbm, idx_hbm, buf_v, vals_v, idxs_v):
    c = lax.axis_index("c")
    s = lax.axis_index("s")
    pltpu.sync_copy(nms_hbm.at[c, pl.ds(s * _SEG, _SEG)], buf_v)

    def fill(j, _):
        vals_v[pl.ds(j * 16, 16)] = jnp.full((16,), -1.0, jnp.float32)
        idxs_v[pl.ds(j * 16, 16)] = jnp.zeros((16,), jnp.int32)
        return 0

    lax.fori_loop(0, (_CAP + 16) // 16, fill, 0)
    lane = lax.iota(jnp.int32, 16)
    one = jnp.full((16,), 1, jnp.int32)
    zero = jnp.zeros((16,), jnp.int32)
    lane_base = lane * _CAPL
    gbase = lane * _LANE_SEG
    base0 = s * _SEG

    def step(i, cnt):
        gidx = gbase + i
        v = plsc.load_gather(buf_v, [gidx])
        fidx = base0 + gidx
        m = v != 0.0
        dst = jnp.where(m & (cnt < _CAPL), lane_base + cnt, _CAP)
        plsc.store_scatter(vals_v, [dst], v)
        plsc.store_scatter(idxs_v, [dst], fidx)
        return cnt + jnp.where(m, one, zero)

    lax.fori_loop(0, _LANE_SEG, step, zero)
    pltpu.sync_copy(vals_v.at[pl.ds(0, _CAP)], vals_hbm.at[c, s])
    pltpu.sync_copy(idxs_v.at[pl.ds(0, _CAP)], idx_hbm.at[c, s])


def _compact_sc(nms_flat):
    b = nms_flat.shape[0]
    run = functools.partial(
        pl.kernel,
        mesh=plsc.VectorSubcoreMesh(core_axis_name="c", subcore_axis_name="s"),
        compiler_params=pltpu.CompilerParams(needs_layout_passes=False),
        out_type=(
            jax.ShapeDtypeStruct((b, _N_SUB, _CAP), jnp.float32),
            jax.ShapeDtypeStruct((b, _N_SUB, _CAP), jnp.int32),
        ),
        scratch_types=[
            pltpu.VMEM((_SEG,), jnp.float32),
            pltpu.VMEM((_CAP + 16,), jnp.float32),
            pltpu.VMEM((_CAP + 16,), jnp.int32),
        ],
    )(_compact_body)
    return run(nms_flat)


_N_KPT = _TOP_K
_CH = 96
_CH_PER_SUB = _CH // _N_SUB


def _desc_gather_body(dm_hbm, idx_hbm, out_hbm, idx_v, absidx_v, row_v, sem):
    c = lax.axis_index("c")
    s = lax.axis_index("s")
    pltpu.sync_copy(idx_hbm.at[c], idx_v)
    for ch in range(_CH_PER_SUB):
        chan = s * _CH_PER_SUB + ch
        base = (c * _CH + chan) * (_H * _W)

        def add(j, _):
            absidx_v[pl.ds(j * 16, 16)] = idx_v[pl.ds(j * 16, 16)] + base
            return 0

        lax.fori_loop(0, _N_KPT // 16, add, 0)
        pltpu.async_copy(dm_hbm.at[absidx_v], row_v, sem).wait()
        pltpu.sync_copy(row_v, out_hbm.at[c, chan])


def _desc_gather_sc(dm_flat, gidx):
    b = gidx.shape[0]
    run = functools.partial(
        pl.kernel,
        mesh=plsc.VectorSubcoreMesh(core_axis_name="c", subcore_axis_name="s"),
        compiler_params=pltpu.CompilerParams(needs_layout_passes=False),
        out_type=jax.ShapeDtypeStruct((b, _CH, _N_KPT), jnp.float32),
        scratch_types=[
            pltpu.VMEM((_N_KPT,), jnp.int32),
            pltpu.VMEM((_N_KPT,), jnp.int32),
            pltpu.VMEM((_N_KPT,), jnp.float32),
            pltpu.SemaphoreType.DMA,
        ],
    )(_desc_gather_body)
    return run(dm_flat, gidx)


def _grid_sample_bilinear(img, kxy):
    H, W = img.shape
    x = (kxy[:, 0] + 1.0) * 0.5 * (W - 1)
    y = (kxy[:, 1] + 1.0) * 0.5 * (H - 1)
    x0 = jnp.floor(x)
    y0 = jnp.floor(y)
    wx1 = x - x0
    wx0 = 1.0 - wx1
    wy1 = y - y0
    wy0 = 1.0 - wy1
    x0i = jnp.clip(x0, 0, W - 1).astype(jnp.int32)
    x1i = jnp.clip(x0 + 1, 0, W - 1).astype(jnp.int32)
    y0i = jnp.clip(y0, 0, H - 1).astype(jnp.int32)
    y1i = jnp.clip(y0 + 1, 0, H - 1).astype(jnp.int32)
    return (wy0 * wx0 * img[y0i, x0i] + wy0 * wx1 * img[y0i, x1i]
            + wy1 * wx0 * img[y1i, x0i] + wy1 * wx1 * img[y1i, x1i])


def kernel(scores_map, descriptor_map):
    b, _, h, w = scores_map.shape
    nms = _nms_pallas(scores_map)
    cvals, cidx = _compact_sc(nms.reshape(b, -1))
    fill_v = jnp.zeros((b, _FILL), jnp.float32)
    fill_i = jnp.broadcast_to(jnp.arange(_FILL, dtype=jnp.int32), (b, _FILL))
    allv = jnp.concatenate([fill_v, cvals.reshape(b, -1)], axis=1)
    alli = jnp.concatenate([fill_i, cidx.reshape(b, -1)], axis=1)
    _, srt = lax.sort((-allv, alli), dimension=1, num_keys=1, is_stable=True)
    idx = srt[:, :_TOP_K]
    kx = (idx % w).astype(jnp.float32)
    ky = (idx // w).astype(jnp.float32)
    kxy = jnp.stack([kx, ky], axis=-1)
    denom = jnp.array([w - 1, h - 1], dtype=jnp.float32)
    kxy = kxy / denom * 2.0 - 1.0
    kptscores = jax.vmap(_grid_sample_bilinear)(scores_map[:, 0], kxy)

    B, C, H, W = descriptor_map.shape
    scale = jnp.array([W - 1, H - 1], dtype=jnp.float32)
    ki = ((kxy + 1.0) / 2.0 * scale).astype(jnp.int32)
    gidx = ki[:, :, 1] * W + ki[:, :, 0]
    d = _desc_gather_sc(descriptor_map.reshape(-1), gidx)
    n = jnp.sqrt(jnp.sum(d * d, axis=1, keepdims=True))
    d = d / jnp.maximum(n, 1e-12)
    descriptors = jnp.transpose(d, (0, 2, 1))
    return kxy, descriptors, kptscores
